# Initial kernel scaffold; baseline (speedup 1.0000x reference)
import jax
import jax.numpy as jnp
from jax.experimental import pallas as pl


def kernel(input_ids, embedding_matrix):
    raise NotImplementedError("write your pallas kernel here")


# speedup vs baseline: 6.3764x; 6.3764x over previous
import functools

import jax
import jax.numpy as jnp
from jax import lax
from jax.experimental import pallas as pl
from jax.experimental.pallas import tpu as pltpu
from jax.experimental.pallas import tpu_sc as plsc

D = 128
ROWS = 4096 * 200
NW = 32
PER_W = ROWS // NW
CHUNK = 128
NCH = PER_W // CHUNK

_mesh = plsc.VectorSubcoreMesh(core_axis_name="c", subcore_axis_name="s")


@functools.partial(
    pl.kernel,
    mesh=_mesh,
    out_type=jax.ShapeDtypeStruct((ROWS, D), jnp.float32),
    scratch_types=[
        pltpu.VMEM((NCH, CHUNK), jnp.int32),
        pltpu.VMEM((CHUNK, D), jnp.float32),
        pltpu.SemaphoreType.DMA,
    ],
)
def _gather_kernel(idx_hbm, table_hbm, out_hbm, idx_v, rows_v, sem):
    wid = lax.axis_index("s") * 2 + lax.axis_index("c")
    pltpu.sync_copy(idx_hbm.at[wid], idx_v)

    def body(j, carry):
        pltpu.async_copy(table_hbm.at[idx_v.at[j]], rows_v, sem).wait()
        base = (wid * NCH + j) * CHUNK
        pltpu.sync_copy(rows_v, out_hbm.at[pl.ds(base, CHUNK)])
        return carry

    lax.fori_loop(0, NCH, body, 0)


def kernel(input_ids, embedding_matrix):
    idx = input_ids.reshape(NW, NCH, CHUNK).astype(jnp.int32)
    out = _gather_kernel(idx, embedding_matrix)
    return out.reshape(4096, 200, D)


# speedup vs baseline: 9.2142x; 1.4450x over previous
import functools

import jax
import jax.numpy as jnp
from jax import lax
from jax.experimental import pallas as pl
from jax.experimental.pallas import tpu as pltpu
from jax.experimental.pallas import tpu_sc as plsc

D = 128
ROWS = 4096 * 200
NW = 32
PER_W = ROWS // NW
CHUNK = 128
NCH = PER_W // CHUNK
NBUF = 4
LAG = 2

_mesh = plsc.VectorSubcoreMesh(core_axis_name="c", subcore_axis_name="s")


@functools.partial(
    pl.kernel,
    mesh=_mesh,
    out_type=jax.ShapeDtypeStruct((ROWS, D), jnp.float32),
    scratch_types=(
        [pltpu.VMEM((NCH, CHUNK), jnp.int32)]
        + [pltpu.VMEM((CHUNK, D), jnp.float32)] * NBUF
        + [pltpu.SemaphoreType.DMA] * (2 * NBUF)
    ),
)
def _gather_kernel(idx_hbm, table_hbm, out_hbm, idx_v, *rest):
    rows = rest[:NBUF]
    gsem = rest[NBUF:2 * NBUF]
    ssem = rest[2 * NBUF:]
    wid = lax.axis_index("s") * 2 + lax.axis_index("c")
    pltpu.sync_copy(idx_hbm.at[wid], idx_v)
    out_base = wid * NCH

    def start_gather(j, b):
        pltpu.async_copy(table_hbm.at[idx_v.at[j]], rows[b], gsem[b])

    def wait_gather(j, b):
        pltpu.make_async_copy(table_hbm.at[idx_v.at[j]], rows[b], gsem[b]).wait()

    def start_store(j, b):
        pltpu.async_copy(
            rows[b], out_hbm.at[pl.ds((out_base + j) * CHUNK, CHUNK)], ssem[b])

    def wait_store(b):
        pltpu.make_async_copy(
            rows[b], out_hbm.at[pl.ds(out_base * CHUNK, CHUNK)], ssem[b]).wait()

    for b in range(NBUF):
        start_gather(b, b)
        if b >= LAG:
            wait_gather(b - LAG, b - LAG)
            start_store(b - LAG, b - LAG)

    def body(g, carry):
        for b in range(NBUF):
            j = g * NBUF + b
            wait_store(b)
            start_gather(j, b)
            bb = (b - LAG) % NBUF
            wait_gather(j - LAG, bb)
            start_store(j - LAG, bb)
        return carry

    lax.fori_loop(1, NCH // NBUF, body, 0)

    for t in range(LAG):
        j = NCH - LAG + t
        b = j % NBUF
        wait_gather(j, b)
        start_store(j, b)
    for b in range(NBUF):
        wait_store(b)


def kernel(input_ids, embedding_matrix):
    idx = input_ids.reshape(NW, NCH, CHUNK).astype(jnp.int32)
    out = _gather_kernel(idx, embedding_matrix)
    return out.reshape(4096, 200, D)
